# baseline (speedup 1.0000x reference)
import jax
import jax.numpy as jnp
from jax import lax
from jax.experimental import pallas as pl
from jax.experimental.pallas import tpu as pltpu
from jax.experimental.pallas import tpu_sc as plsc

N_CHANNELS = 4096
N_ENERGIES = 16384
SC_ROWS = 512
TC_ROWS = N_CHANNELS - SC_ROWS
BC = 128

NSC, NSS, LANES = 2, 16, 16
NW = NSC * NSS
RPW = SC_ROWS // NW
CHUNK = 4


def _flux_expr(params_ref, en_ref):
    alpha = params_ref[0] + 1.2
    norm = params_ref[1]
    one_m_a = 1.0 - alpha
    e_low = en_ref[0:1, :]
    e_high = en_ref[1:2, :]
    return norm * (
        jnp.power(e_high, one_m_a) - jnp.power(e_low, one_m_a)
    ) / one_m_a


def _flux_body(params_ref, en_ref, out_ref):
    out_ref[...] = _flux_expr(params_ref, en_ref)


def _tc_body(params_ref, en_ref, tm_ref, out_ref, flux_ref):
    @pl.when(pl.program_id(0) == 0)
    def _():
        flux_ref[...] = _flux_expr(params_ref, en_ref)

    flux = flux_ref[...]
    acc = jnp.sum(tm_ref[...] * flux, axis=1)
    out_ref[...] = jnp.maximum(acc, 1e-6)


def _sc_body(tm_hbm, flux_hbm, out_hbm, flux_v, rows_v, out_v):
    c = lax.axis_index("c")
    s = lax.axis_index("s")
    wid = s * NSC + c
    pltpu.sync_copy(flux_hbm, flux_v)
    row_base = TC_ROWS + wid * RPW
    lane = lax.broadcasted_iota(jnp.int32, (LANES,), 0)
    outvec = jnp.zeros((LANES,), jnp.float32)
    for chunk in range(RPW // CHUNK):
        pltpu.sync_copy(
            tm_hbm.at[pl.ds(row_base + chunk * CHUNK, CHUNK)], rows_v)
        for r in range(CHUNK):
            def kbody(k, acc, _r=r):
                off = k * LANES
                return acc + (rows_v[_r, pl.ds(off, LANES)]
                              * flux_v[pl.ds(off, LANES)])
            acc = lax.fori_loop(0, N_ENERGIES // LANES, kbody,
                                jnp.zeros((LANES,), jnp.float32), unroll=8)
            for sh in (1, 2, 4, 8):
                acc = acc + acc.at[lane ^ sh].get(mode="promise_in_bounds")
            outvec = jnp.where(lane == (chunk * CHUNK + r), acc, outvec)
    out_v[...] = jnp.maximum(outvec, 1e-6)
    pltpu.sync_copy(out_v, out_hbm.at[pl.ds(wid * LANES, LANES)])


def kernel(parameters, energies, transfer_matrix):
    flux2d = pl.pallas_call(
        _flux_body,
        grid=(1,),
        in_specs=[
            pl.BlockSpec(memory_space=pltpu.SMEM),
            pl.BlockSpec((2, N_ENERGIES), lambda i: (0, 0)),
        ],
        out_specs=pl.BlockSpec((1, N_ENERGIES), lambda i: (0, 0)),
        out_shape=jax.ShapeDtypeStruct((1, N_ENERGIES), jnp.float32),
    )(parameters, energies)
    flux1d = flux2d.reshape(N_ENERGIES)

    sc_mv = pl.kernel(
        _sc_body,
        out_type=jax.ShapeDtypeStruct((SC_ROWS,), jnp.float32),
        mesh=plsc.VectorSubcoreMesh(
            core_axis_name="c", subcore_axis_name="s",
            num_cores=NSC, num_subcores=NSS),
        scratch_types=[
            pltpu.VMEM((N_ENERGIES,), jnp.float32),
            pltpu.VMEM((CHUNK, N_ENERGIES), jnp.float32),
            pltpu.VMEM((LANES,), jnp.float32),
        ],
    )
    out_tail = sc_mv(transfer_matrix, flux1d)

    out_head = pl.pallas_call(
        _tc_body,
        grid=(TC_ROWS // BC,),
        in_specs=[
            pl.BlockSpec(memory_space=pltpu.SMEM),
            pl.BlockSpec((2, N_ENERGIES), lambda i: (0, 0)),
            pl.BlockSpec((BC, N_ENERGIES), lambda i: (i, 0)),
        ],
        out_specs=pl.BlockSpec((BC,), lambda i: (i,)),
        out_shape=jax.ShapeDtypeStruct((TC_ROWS,), jnp.float32),
        scratch_shapes=[pltpu.VMEM((1, N_ENERGIES), jnp.float32)],
    )(parameters, energies, transfer_matrix)

    return jnp.concatenate([out_head, out_tail])


# speedup vs baseline: 1.0123x; 1.0123x over previous
import jax
import jax.numpy as jnp
from jax.experimental import pallas as pl
from jax.experimental.pallas import tpu as pltpu

N_CHANNELS = 4096
N_ENERGIES = 16384
BC = 128
BE = 8192


def _body(params_ref, en_ref, tm_ref, out_ref, flux_ref, acc_ref):
    j = pl.program_id(1)

    @pl.when((pl.program_id(0) == 0) & (j == 0))
    def _():
        alpha = params_ref[0] + 1.2
        norm = params_ref[1]
        one_m_a = 1.0 - alpha
        e_low = en_ref[0:1, :]
        e_high = en_ref[1:2, :]
        flux_ref[...] = norm * (
            jnp.power(e_high, one_m_a) - jnp.power(e_low, one_m_a)
        ) / one_m_a

    flux = flux_ref[:, pl.ds(j * BE, BE)]
    part = jnp.sum(tm_ref[...] * flux, axis=1)

    @pl.when(j == 0)
    def _():
        acc_ref[...] = part

    @pl.when(j > 0)
    def _():
        acc_ref[...] += part

    @pl.when(j == (N_ENERGIES // BE) - 1)
    def _():
        out_ref[...] = jnp.maximum(acc_ref[...], 1e-6)


def kernel(parameters, energies, transfer_matrix):
    out = pl.pallas_call(
        _body,
        grid=(N_CHANNELS // BC, N_ENERGIES // BE),
        in_specs=[
            pl.BlockSpec(memory_space=pltpu.SMEM),
            pl.BlockSpec((2, N_ENERGIES), lambda i, j: (0, 0)),
            pl.BlockSpec((BC, BE), lambda i, j: (i, j)),
        ],
        out_specs=pl.BlockSpec((BC,), lambda i, j: (i,)),
        out_shape=jax.ShapeDtypeStruct((N_CHANNELS,), jnp.float32),
        scratch_shapes=[
            pltpu.VMEM((1, N_ENERGIES), jnp.float32),
            pltpu.VMEM((BC,), jnp.float32),
        ],
    )(parameters, energies, transfer_matrix)
    return out


# speedup vs baseline: 1.0146x; 1.0023x over previous
import jax
import jax.numpy as jnp
from jax.experimental import pallas as pl
from jax.experimental.pallas import tpu as pltpu

N_CHANNELS = 4096
N_ENERGIES = 16384
BC = 64


def _body(params_ref, en_ref, tm_ref, out_ref, flux_ref):
    i = pl.program_id(0)

    @pl.when(i == 0)
    def _():
        alpha = params_ref[0] + 1.2
        norm = params_ref[1]
        one_m_a = 1.0 - alpha
        e_low = en_ref[0:1, :]
        e_high = en_ref[1:2, :]
        flux_ref[...] = norm * (
            jnp.power(e_high, one_m_a) - jnp.power(e_low, one_m_a)
        ) / one_m_a

    flux = flux_ref[...]
    acc = jnp.sum(tm_ref[...] * flux, axis=1)
    out_ref[...] = jnp.maximum(acc, 1e-6).reshape(1, 1, BC)


def kernel(parameters, energies, transfer_matrix):
    out = pl.pallas_call(
        _body,
        grid=(N_CHANNELS // BC,),
        in_specs=[
            pl.BlockSpec(memory_space=pltpu.SMEM),
            pl.BlockSpec((2, N_ENERGIES), lambda i: (0, 0)),
            pl.BlockSpec((BC, N_ENERGIES), lambda i: (i, 0)),
        ],
        out_specs=pl.BlockSpec((1, 1, BC), lambda i: (i, 0, 0)),
        out_shape=jax.ShapeDtypeStruct((N_CHANNELS // BC, 1, BC), jnp.float32),
        scratch_shapes=[pltpu.VMEM((1, N_ENERGIES), jnp.float32)],
    )(parameters, energies, transfer_matrix)
    return out.reshape(N_CHANNELS)


# speedup vs baseline: 1.1985x; 1.1813x over previous
import jax
import jax.numpy as jnp
from jax.experimental import pallas as pl
from jax.experimental.pallas import tpu as pltpu

N_CHANNELS = 4096
N_ENERGIES = 16384
BC = 128


def _body(params_ref, en_ref, tm_ref, out_ref, flux_ref):
    i = pl.program_id(0)

    @pl.when(i == 0)
    def _():
        alpha = params_ref[0] + 1.2
        norm = params_ref[1]
        one_m_a = 1.0 - alpha
        e_low = en_ref[0:1, :]
        e_high = en_ref[1:2, :]
        flux_ref[...] = norm * (
            jnp.power(e_high, one_m_a) - jnp.power(e_low, one_m_a)
        ) / one_m_a

    flux = flux_ref[...]
    acc = jnp.sum(tm_ref[...] * flux, axis=1)
    out_ref[...] = jnp.maximum(acc, 1e-6).reshape(1, 1, BC)


def kernel(parameters, energies, transfer_matrix):
    out = pl.pallas_call(
        _body,
        grid=(N_CHANNELS // BC,),
        in_specs=[
            pl.BlockSpec(memory_space=pltpu.SMEM),
            pl.BlockSpec((2, N_ENERGIES), lambda i: (0, 0)),
            pl.BlockSpec((BC, N_ENERGIES), lambda i: (i, 0)),
        ],
        out_specs=pl.BlockSpec((1, 1, BC), lambda i: (i, 0, 0)),
        out_shape=jax.ShapeDtypeStruct((N_CHANNELS // BC, 1, BC), jnp.float32),
        scratch_shapes=[pltpu.VMEM((1, N_ENERGIES), jnp.float32)],
    )(parameters, energies, transfer_matrix)
    return out.reshape(N_CHANNELS)
